# baseline (speedup 1.0000x reference)
import jax
import jax.numpy as jnp
from jax import lax
from jax.experimental import pallas as pl
from jax.experimental.pallas import tpu as pltpu
from jax.experimental.pallas import tpu_sc as plsc

N_NODES = 10000
N_EDGES = 320000
D_IN = 128
E_DIM = 16
D_OUT = 128

NUM_CORES = 2
NUM_SUBCORES = 16
NW = NUM_CORES * NUM_SUBCORES
E_PER_W = N_EDGES // NW
CHUNK = 2000
N_CHUNKS = E_PER_W // CHUNK
IB = 80
IB_PER_CHUNK = CHUNK // IB
IB_PER_W = E_PER_W // IB
N_PAD = 10240
ROWS_PER_SUB = N_PAD // NUM_SUBCORES

TBLK_E = 80000
TSEG = TBLK_E // 8
TGRID = N_EDGES // TBLK_E
PSEG = E_PER_W // 8


def _tr_body(in_ref, out_ref):
  x = in_ref[...]
  y = jnp.concatenate([x[:, g * TSEG:(g + 1) * TSEG] for g in range(8)],
                      axis=0)
  out_ref[...] = jnp.transpose(y)


def _tc_transpose(efT):
  return pl.pallas_call(
      _tr_body,
      grid=(TGRID,),
      in_specs=[pl.BlockSpec((E_DIM, TBLK_E), lambda i: (0, i))],
      out_specs=pl.BlockSpec((TSEG, 128), lambda i: (i, 0)),
      out_shape=jax.ShapeDtypeStruct((N_EDGES // 8, 128), jnp.float32),
  )(efT)


def _sc_aggregate(ef2d, dst3d, zeros_in, zeros1_in):
  mesh = plsc.VectorSubcoreMesh(core_axis_name="c", subcore_axis_name="s",
                                num_cores=NUM_CORES,
                                num_subcores=NUM_SUBCORES)

  @pl.kernel(
      out_type=(
          jax.ShapeDtypeStruct((NUM_CORES, N_PAD, E_DIM), jnp.float32),
          jax.ShapeDtypeStruct((NUM_CORES, N_PAD), jnp.float32),
      ),
      mesh=mesh,
      compiler_params=pltpu.CompilerParams(use_tc_tiling_on_sc=False,
                                           needs_layout_passes=False),
      scratch_types=[
          pltpu.VMEM_SHARED((N_PAD, E_DIM), jnp.float32),
          pltpu.VMEM_SHARED((N_PAD,), jnp.float32),
          pltpu.VMEM((CHUNK, E_DIM), jnp.float32),
          pltpu.VMEM((8, PSEG), jnp.int32),
          pltpu.VMEM((E_PER_W,), jnp.int32),
          pltpu.VMEM((112,), jnp.float32),
      ],
  )
  def k(ef_hbm, dst_hbm, zeros_hbm, zeros1_hbm, hs_hbm, dg_hbm,
        acc, dacc, rows_v, idxsrc_v, idx_v, ones_v):
    c = lax.axis_index("c")
    s = lax.axis_index("s")
    wid = c * NUM_SUBCORES + s
    iblk = wid // 8
    wp = wid % 8
    lane = jax.lax.iota(jnp.int32, 16)

    r0 = s * ROWS_PER_SUB
    pltpu.sync_copy(zeros_hbm, acc.at[pl.ds(r0, ROWS_PER_SUB), :])

    pltpu.sync_copy(zeros1_hbm, dacc.at[pl.ds(r0, ROWS_PER_SUB)])

    for i in range(7):
      ones_v[pl.ds(i * 16, 16)] = jnp.full((16,), 1.0, jnp.float32)

    for g in range(8):
      pltpu.sync_copy(dst_hbm.at[iblk, g, wp], idxsrc_v.at[g])

    row_c = lane % 8
    col_c = lane // 8

    @pl.loop(0, E_PER_W // 16)
    def _(t):
      v = plsc.load_gather(idxsrc_v, [row_c, col_c + 2 * t])
      idx_v[pl.ds(16 * t, 16)] = v

    plsc.subcore_barrier()

    @pl.loop(0, N_CHUNKS)
    def _(kk):
      base = wid * E_PER_W + kk * CHUNK
      pltpu.sync_copy(ef_hbm.at[pl.ds(base, CHUNK), :], rows_v)

      @pl.loop(0, IB_PER_CHUNK)
      def _(j):
        idx = idx_v.at[pl.ds(kk * CHUNK + j * IB, IB)]
        pltpu.sync_copy(rows_v.at[pl.ds(j * IB, IB), :], acc.at[idx],
                        add=True)
        pltpu.sync_copy(ones_v.at[pl.ds(0, IB)], dacc.at[idx], add=True)

    plsc.subcore_barrier()

    pltpu.sync_copy(acc.at[pl.ds(r0, ROWS_PER_SUB), :],
                    hs_hbm.at[c, pl.ds(r0, ROWS_PER_SUB), :])
    pltpu.sync_copy(dacc.at[pl.ds(r0, ROWS_PER_SUB)],
                    dg_hbm.at[c, pl.ds(r0, ROWS_PER_SUB)])

  return k(ef2d, dst3d, zeros_in, zeros1_in)


ROW_BLK = 1000


def _nfw_body(nf_ref, wn_ref, b_ref, out_ref):
  out_ref[...] = jnp.dot(nf_ref[...], wn_ref[...],
                         preferred_element_type=jnp.float32) + b_ref[...]


def _tc_nfw(nf2d, wn, bias2d):
  return pl.pallas_call(
      _nfw_body,
      grid=(N_NODES // ROW_BLK,),
      in_specs=[
          pl.BlockSpec((ROW_BLK, D_IN), lambda i: (i, 0)),
          pl.BlockSpec((D_IN, D_OUT), lambda i: (0, 0)),
          pl.BlockSpec((1, D_OUT), lambda i: (0, 0)),
      ],
      out_specs=pl.BlockSpec((ROW_BLK, D_OUT), lambda i: (i, 0)),
      out_shape=jax.ShapeDtypeStruct((N_NODES, D_OUT), jnp.float32),
  )(nf2d, wn, bias2d)


def _mix_body(nfw_ref, hs_ref, dg_ref, we_ref, out_ref):
  hs = hs_ref[0] + hs_ref[1]
  dg = dg_ref[0] + dg_ref[1]
  hn = hs / jnp.maximum(dg, 1.0)
  acc = nfw_ref[...] + jnp.dot(hn, we_ref[...],
                               preferred_element_type=jnp.float32)
  out_ref[...] = jnp.maximum(acc, 0.0)


def _tc_mix(nfw, hsum, deg, we):
  return pl.pallas_call(
      _mix_body,
      grid=(N_NODES // ROW_BLK,),
      in_specs=[
          pl.BlockSpec((ROW_BLK, D_OUT), lambda i: (i, 0)),
          pl.BlockSpec((NUM_CORES, ROW_BLK, E_DIM), lambda i: (0, i, 0)),
          pl.BlockSpec((NUM_CORES, ROW_BLK, 1), lambda i: (0, i, 0)),
          pl.BlockSpec((E_DIM, D_OUT), lambda i: (0, 0)),
      ],
      out_specs=pl.BlockSpec((ROW_BLK, D_OUT), lambda i: (i, 0)),
      out_shape=jax.ShapeDtypeStruct((N_NODES, D_OUT), jnp.float32),
  )(nfw, hsum, deg, we)


def kernel(nfeats, efeats, edge_index, W_apply_w, W_apply_b):
  nf2d = nfeats.reshape(N_NODES, D_IN)
  efT = jnp.transpose(efeats[:, 0, :])
  eflin = _tc_transpose(efT)
  ef2d = eflin.reshape(N_EDGES, E_DIM)
  dst4d = edge_index[1].reshape(TGRID, 8, 8, PSEG)
  zeros_in = jnp.zeros((ROWS_PER_SUB, E_DIM), jnp.float32)
  zeros1_in = jnp.zeros((ROWS_PER_SUB,), jnp.float32)

  hsum, deg = _sc_aggregate(ef2d, dst4d, zeros_in, zeros1_in)
  deg = deg.reshape(NUM_CORES, N_PAD, 1)

  wn = W_apply_w[:, :D_IN].T
  we = W_apply_w[:, D_IN:].T
  bias2d = W_apply_b.reshape(1, D_OUT)
  nfw = _tc_nfw(nf2d, wn, bias2d)
  out2d = _tc_mix(nfw, hsum, deg, we)
  return out2d.reshape(N_NODES, 1, D_OUT)


# speedup vs baseline: 1.1502x; 1.1502x over previous
import jax
import jax.numpy as jnp
from jax import lax
from jax.experimental import pallas as pl
from jax.experimental.pallas import tpu as pltpu
from jax.experimental.pallas import tpu_sc as plsc

N_NODES = 10000
N_EDGES = 320000
D_IN = 128
E_DIM = 16
D_OUT = 128

NUM_CORES = 2
NUM_SUBCORES = 16
NW = NUM_CORES * NUM_SUBCORES
E_PER_W = N_EDGES // NW
CHUNK = 2000
N_CHUNKS = E_PER_W // CHUNK
IB = 80
IB_PER_CHUNK = CHUNK // IB
IB_PER_W = E_PER_W // IB
N_PAD = 10240
ROWS_PER_SUB = N_PAD // NUM_SUBCORES

TBLK_E = 80000
TSEG = TBLK_E // 8
TGRID = N_EDGES // TBLK_E
PSEG = E_PER_W // 8


def _tr_body(in_ref, out_ref):
  x = in_ref[...]
  y = jnp.concatenate([x[:, g * TSEG:(g + 1) * TSEG] for g in range(8)],
                      axis=0)
  out_ref[...] = jnp.transpose(y)


def _tc_transpose(efT):
  return pl.pallas_call(
      _tr_body,
      grid=(TGRID,),
      in_specs=[pl.BlockSpec((E_DIM, TBLK_E), lambda i: (0, i))],
      out_specs=pl.BlockSpec((TSEG, 128), lambda i: (i, 0)),
      out_shape=jax.ShapeDtypeStruct((N_EDGES // 8, 128), jnp.float32),
  )(efT)


def _sc_aggregate(ef2d, dst3d, zeros_in, zeros1_in):
  mesh = plsc.VectorSubcoreMesh(core_axis_name="c", subcore_axis_name="s",
                                num_cores=NUM_CORES,
                                num_subcores=NUM_SUBCORES)

  @pl.kernel(
      out_type=(
          jax.ShapeDtypeStruct((NUM_CORES, N_PAD, E_DIM), jnp.float32),
          jax.ShapeDtypeStruct((NUM_CORES, N_PAD), jnp.float32),
      ),
      mesh=mesh,
      compiler_params=pltpu.CompilerParams(use_tc_tiling_on_sc=False,
                                           needs_layout_passes=False),
      scratch_types=[
          pltpu.VMEM_SHARED((N_PAD, E_DIM), jnp.float32),
          pltpu.VMEM_SHARED((N_PAD,), jnp.float32),
          pltpu.VMEM((CHUNK, E_DIM), jnp.float32),
          pltpu.VMEM((CHUNK, E_DIM), jnp.float32),
          pltpu.VMEM((8, PSEG), jnp.int32),
          pltpu.VMEM((E_PER_W,), jnp.int32),
          pltpu.VMEM((112,), jnp.float32),
          pltpu.VMEM((CHUNK * (E_DIM + 1) // E_DIM, E_DIM),
                     jnp.float32),
          pltpu.SemaphoreType.DMA,
      ],
  )
  def k(ef_hbm, dst_hbm, zeros_hbm, zeros1_hbm, hs_hbm, dg_hbm,
        acc, dacc, rows_a, rows_b, idxsrc_v, idx_v, ones_v, dummy_v, sem):
    c = lax.axis_index("c")
    s = lax.axis_index("s")
    wid = c * NUM_SUBCORES + s
    iblk = wid // 8
    wp = wid % 8
    lane = jax.lax.iota(jnp.int32, 16)

    r0 = s * ROWS_PER_SUB
    pltpu.sync_copy(zeros_hbm, acc.at[pl.ds(r0, ROWS_PER_SUB), :])

    pltpu.sync_copy(zeros1_hbm, dacc.at[pl.ds(r0, ROWS_PER_SUB)])

    for i in range(7):
      ones_v[pl.ds(i * 16, 16)] = jnp.full((16,), 1.0, jnp.float32)

    for g in range(8):
      pltpu.sync_copy(dst_hbm.at[iblk, g, wp], idxsrc_v.at[g])

    row_c = lane % 8
    col_c = lane // 8

    @pl.loop(0, E_PER_W // 16)
    def _(t):
      v = plsc.load_gather(idxsrc_v, [row_c, col_c + 2 * t])
      idx_v[pl.ds(16 * t, 16)] = v

    plsc.subcore_barrier()

    def stage(buf, kk):
      base = wid * E_PER_W + kk * CHUNK
      pltpu.sync_copy(ef_hbm.at[pl.ds(base, CHUNK), :], buf)

    def fire(buf, kk):
      @pl.loop(0, IB_PER_CHUNK)
      def _(j):
        idx = idx_v.at[pl.ds(kk * CHUNK + j * IB, IB)]
        pltpu.async_copy(buf.at[pl.ds(j * IB, IB), :], acc.at[idx], sem,
                         add=True)
        pltpu.async_copy(ones_v.at[pl.ds(0, IB)], dacc.at[idx], sem,
                         add=True)

    def drain():
      pltpu.make_async_copy(ef_hbm.at[pl.ds(0, CHUNK * 17 // 16), :],
                            dummy_v, sem).wait()

    stage(rows_a, 0)
    for kk in range(N_CHUNKS):
      buf = rows_a if kk % 2 == 0 else rows_b
      nxt = rows_b if kk % 2 == 0 else rows_a
      fire(buf, kk)
      if kk + 1 < N_CHUNKS:
        stage(nxt, kk + 1)
      drain()

    plsc.subcore_barrier()

    pltpu.sync_copy(acc.at[pl.ds(r0, ROWS_PER_SUB), :],
                    hs_hbm.at[c, pl.ds(r0, ROWS_PER_SUB), :])
    pltpu.sync_copy(dacc.at[pl.ds(r0, ROWS_PER_SUB)],
                    dg_hbm.at[c, pl.ds(r0, ROWS_PER_SUB)])

  return k(ef2d, dst3d, zeros_in, zeros1_in)


ROW_BLK = 1000


def _nfw_body(nf_ref, wn_ref, b_ref, out_ref):
  out_ref[...] = jnp.dot(nf_ref[...], wn_ref[...],
                         preferred_element_type=jnp.float32) + b_ref[...]


def _tc_nfw(nf2d, wn, bias2d):
  return pl.pallas_call(
      _nfw_body,
      grid=(N_NODES // ROW_BLK,),
      in_specs=[
          pl.BlockSpec((ROW_BLK, D_IN), lambda i: (i, 0)),
          pl.BlockSpec((D_IN, D_OUT), lambda i: (0, 0)),
          pl.BlockSpec((1, D_OUT), lambda i: (0, 0)),
      ],
      out_specs=pl.BlockSpec((ROW_BLK, D_OUT), lambda i: (i, 0)),
      out_shape=jax.ShapeDtypeStruct((N_NODES, D_OUT), jnp.float32),
  )(nf2d, wn, bias2d)


def _mix_body(nfw_ref, hs_ref, dg_ref, we_ref, out_ref):
  hs = hs_ref[0] + hs_ref[1]
  dg = dg_ref[0] + dg_ref[1]
  hn = hs / jnp.maximum(dg, 1.0)
  acc = nfw_ref[...] + jnp.dot(hn, we_ref[...],
                               preferred_element_type=jnp.float32)
  out_ref[...] = jnp.maximum(acc, 0.0)


def _tc_mix(nfw, hsum, deg, we):
  return pl.pallas_call(
      _mix_body,
      grid=(N_NODES // ROW_BLK,),
      in_specs=[
          pl.BlockSpec((ROW_BLK, D_OUT), lambda i: (i, 0)),
          pl.BlockSpec((NUM_CORES, ROW_BLK, E_DIM), lambda i: (0, i, 0)),
          pl.BlockSpec((NUM_CORES, ROW_BLK, 1), lambda i: (0, i, 0)),
          pl.BlockSpec((E_DIM, D_OUT), lambda i: (0, 0)),
      ],
      out_specs=pl.BlockSpec((ROW_BLK, D_OUT), lambda i: (i, 0)),
      out_shape=jax.ShapeDtypeStruct((N_NODES, D_OUT), jnp.float32),
  )(nfw, hsum, deg, we)


def kernel(nfeats, efeats, edge_index, W_apply_w, W_apply_b):
  nf2d = nfeats.reshape(N_NODES, D_IN)
  efT = jnp.transpose(efeats[:, 0, :])
  eflin = _tc_transpose(efT)
  ef2d = eflin.reshape(N_EDGES, E_DIM)
  dst4d = edge_index[1].reshape(TGRID, 8, 8, PSEG)
  zeros_in = jnp.zeros((ROWS_PER_SUB, E_DIM), jnp.float32)
  zeros1_in = jnp.zeros((ROWS_PER_SUB,), jnp.float32)

  hsum, deg = _sc_aggregate(ef2d, dst4d, zeros_in, zeros1_in)
  deg = deg.reshape(NUM_CORES, N_PAD, 1)

  wn = W_apply_w[:, :D_IN].T
  we = W_apply_w[:, D_IN:].T
  bias2d = W_apply_b.reshape(1, D_OUT)
  nfw = _tc_nfw(nf2d, wn, bias2d)
  out2d = _tc_mix(nfw, hsum, deg, we)
  return out2d.reshape(N_NODES, 1, D_OUT)
